# baseline (speedup 1.0000x reference)
import functools

import jax
import jax.numpy as jnp
from jax import lax
from jax.experimental import pallas as pl
from jax.experimental.pallas import tpu as pltpu
from jax.experimental.pallas import tpu_sc as plsc

B, C, T, D = 16, 16, 200, 768
NC, NS = 2, 16
NW = NC * NS
L = 16
PAIRS_W = B * C // NW
ROWS_W = PAIRS_W * T
R = 40
NCHUNKS = ROWS_W // R
CPP = T // R
NBUF = 4


def _sc_body(x_hbm, ids_hbm, tok_hbm, cht_hbm, out_hbm,
             idx_all, ids_v, ch8,
             b0, b1, b2, b3, g0, g1, g2, g3, o0, o1, o2, o3, psem, csem):
    bufs = (b0, b1, b2, b3)
    gsems = (g0, g1, g2, g3)
    osems = (o0, o1, o2, o3)
    wid = lax.axis_index("s") * NC + lax.axis_index("c")
    bi = wid // 2
    c0 = lax.rem(wid, 2) * PAIRS_W
    idx_cp = pltpu.async_copy(x_hbm.at[pl.ds(wid * ROWS_W, ROWS_W)],
                              idx_all, psem)
    pltpu.sync_copy(ids_hbm, ids_v)
    ch_cp = pltpu.async_copy(cht_hbm.at[ids_v.at[pl.ds(c0, PAIRS_W)]], ch8,
                             csem)
    idx_cp.wait()

    def issue_gather(j, b):
        pltpu.async_copy(tok_hbm.at[idx_all.at[pl.ds(j * R, R)]],
                         bufs[b], gsems[b])

    def wait_gather(b):
        pltpu.make_async_copy(tok_hbm.at[idx_all.at[pl.ds(0, R)]],
                              bufs[b], gsems[b]).wait()

    def issue_scatter(j, b):
        pltpu.async_copy(bufs[b],
                         out_hbm.at[pl.ds(wid * ROWS_W + j * R, R)],
                         osems[b])

    def wait_scatter(b):
        pltpu.make_async_copy(bufs[b], out_hbm.at[pl.ds(0, R)],
                              osems[b]).wait()

    def add_channel(j, b):
        cl = j // CPP
        chvs = [ch8[cl, pl.ds(d * L, L)] for d in range(D // L)]
        buf = bufs[b]

        @plsc.parallel_loop(0, R, 1, unroll=2)
        def row_body(r):
            for d in range(D // L):
                buf[r, pl.ds(d * L, L)] += chvs[d]

    def slot(j, b, prefetch, wait_o):
        wait_gather(b)
        if prefetch:
            nb = (b + 2) % NBUF
            if wait_o:
                wait_scatter(nb)
            issue_gather(j + 2, nb)
        add_channel(j, b)
        issue_scatter(j, b)

    issue_gather(0, 0)
    issue_gather(1, 1)
    ch_cp.wait()
    slot(0, 0, True, False)
    slot(1, 1, True, False)
    slot(2, 2, True, True)
    slot(3, 3, True, True)

    def outer(k, carry):
        j0 = 4 * k
        for s in range(4):
            slot(j0 + s, s, True, True)
        return carry

    lax.fori_loop(1, NCHUNKS // 4 - 1, outer, 0, unroll=False)

    n = NCHUNKS
    slot(n - 4, (n - 4) % NBUF, True, True)
    slot(n - 3, (n - 3) % NBUF, True, True)
    slot(n - 2, (n - 2) % NBUF, False, False)
    slot(n - 1, (n - 1) % NBUF, False, False)
    for b in range(NBUF):
        wait_scatter(b)


@jax.jit
def _sc_call(x32, ids32, token_table, channel_table):
    mesh = plsc.VectorSubcoreMesh(core_axis_name="c", subcore_axis_name="s")
    f = pl.kernel(
        _sc_body,
        out_type=jax.ShapeDtypeStruct((B * C * T, D), jnp.float32),
        mesh=mesh,
        scratch_types=[
            pltpu.VMEM((ROWS_W,), jnp.int32),
            pltpu.VMEM((C,), jnp.int32),
            pltpu.VMEM((PAIRS_W, D), jnp.float32),
            pltpu.VMEM((R, D), jnp.float32),
            pltpu.VMEM((R, D), jnp.float32),
            pltpu.VMEM((R, D), jnp.float32),
            pltpu.VMEM((R, D), jnp.float32),
            pltpu.SemaphoreType.DMA,
            pltpu.SemaphoreType.DMA,
            pltpu.SemaphoreType.DMA,
            pltpu.SemaphoreType.DMA,
            pltpu.SemaphoreType.DMA,
            pltpu.SemaphoreType.DMA,
            pltpu.SemaphoreType.DMA,
            pltpu.SemaphoreType.DMA,
            pltpu.SemaphoreType.DMA,
            pltpu.SemaphoreType.DMA,
        ],
    )
    return f(x32, ids32, token_table, channel_table)


def kernel(x, ids, token_table, channel_table):
    out = _sc_call(x.reshape(-1).astype(jnp.int32), ids.astype(jnp.int32),
                   token_table, channel_table)
    return out.reshape(B, C, T, D)


# speedup vs baseline: 1.0338x; 1.0338x over previous
import functools

import jax
import jax.numpy as jnp
from jax import lax
from jax.experimental import pallas as pl
from jax.experimental.pallas import tpu as pltpu
from jax.experimental.pallas import tpu_sc as plsc

B, C, T, D = 16, 16, 200, 768
NC, NS = 2, 16
NW = NC * NS
L = 16
PAIRS_W = B * C // NW
ROWS_W = PAIRS_W * T
R = 40
NCHUNKS = ROWS_W // R
CPP = T // R
NBUF = 4


def _sc_body(x_hbm, ids_hbm, tok_hbm, cht_hbm, out_hbm,
             idx_all, ids_v, ch8,
             b0, b1, b2, b3, g0, g1, g2, g3, o0, o1, o2, o3, psem, csem):
    bufs = (b0, b1, b2, b3)
    gsems = (g0, g1, g2, g3)
    osems = (o0, o1, o2, o3)
    wid = lax.axis_index("s") * NC + lax.axis_index("c")
    bi = wid // 2
    c0 = lax.rem(wid, 2) * PAIRS_W
    idx_cp = pltpu.async_copy(x_hbm.at[pl.ds(wid * ROWS_W, ROWS_W)],
                              idx_all, psem)
    pltpu.sync_copy(ids_hbm, ids_v)
    ch_cp = pltpu.async_copy(cht_hbm.at[ids_v.at[pl.ds(c0, PAIRS_W)]], ch8,
                             csem)
    idx_cp.wait()

    def issue_gather(j, b):
        pltpu.async_copy(tok_hbm.at[idx_all.at[pl.ds(j * R, R)]],
                         bufs[b], gsems[b])

    def wait_gather(b):
        pltpu.make_async_copy(tok_hbm.at[idx_all.at[pl.ds(0, R)]],
                              bufs[b], gsems[b]).wait()

    def issue_scatter(j, b):
        pltpu.async_copy(bufs[b],
                         out_hbm.at[pl.ds(wid * ROWS_W + j * R, R)],
                         osems[b])

    def wait_scatter(b):
        pltpu.make_async_copy(bufs[b], out_hbm.at[pl.ds(0, R)],
                              osems[b]).wait()

    def add_channel(j, b):
        cl = j // CPP
        chvs = [ch8[cl, pl.ds(d * L, L)] for d in range(D // L)]
        buf = bufs[b]

        def row_body(r, rc):
            for d in range(D // L):
                buf[r, pl.ds(d * L, L)] += chvs[d]
            return rc

        lax.fori_loop(0, R, row_body, 0, unroll=False)

    def slot(j, b):
        wait_gather(b)
        nb = (b + 2) % NBUF

        @pl.when(j >= 2)
        def _():
            wait_scatter(nb)

        @pl.when(j + 2 < NCHUNKS)
        def _():
            issue_gather(j + 2, nb)

        add_channel(j, b)
        issue_scatter(j, b)

    issue_gather(0, 0)
    issue_gather(1, 1)
    ch_cp.wait()

    def outer(k, carry):
        j0 = 4 * k
        for s in range(4):
            slot(j0 + s, s)
        return carry

    lax.fori_loop(0, NCHUNKS // 4, outer, 0, unroll=False)

    wait_scatter((NCHUNKS - 2) % NBUF)
    wait_scatter((NCHUNKS - 1) % NBUF)


@jax.jit
def _sc_call(x32, ids32, token_table, channel_table):
    mesh = plsc.VectorSubcoreMesh(core_axis_name="c", subcore_axis_name="s")
    f = pl.kernel(
        _sc_body,
        out_type=jax.ShapeDtypeStruct((B * C * T, D), jnp.float32),
        mesh=mesh,
        scratch_types=[
            pltpu.VMEM((ROWS_W,), jnp.int32),
            pltpu.VMEM((C,), jnp.int32),
            pltpu.VMEM((PAIRS_W, D), jnp.float32),
            pltpu.VMEM((R, D), jnp.float32),
            pltpu.VMEM((R, D), jnp.float32),
            pltpu.VMEM((R, D), jnp.float32),
            pltpu.VMEM((R, D), jnp.float32),
            pltpu.SemaphoreType.DMA,
            pltpu.SemaphoreType.DMA,
            pltpu.SemaphoreType.DMA,
            pltpu.SemaphoreType.DMA,
            pltpu.SemaphoreType.DMA,
            pltpu.SemaphoreType.DMA,
            pltpu.SemaphoreType.DMA,
            pltpu.SemaphoreType.DMA,
            pltpu.SemaphoreType.DMA,
            pltpu.SemaphoreType.DMA,
        ],
    )
    return f(x32, ids32, token_table, channel_table)


def kernel(x, ids, token_table, channel_table):
    out = _sc_call(x.reshape(-1).astype(jnp.int32), ids.astype(jnp.int32),
                   token_table, channel_table)
    return out.reshape(B, C, T, D)


# speedup vs baseline: 1.0350x; 1.0012x over previous
import functools

import jax
import jax.numpy as jnp
from jax import lax
from jax.experimental import pallas as pl
from jax.experimental.pallas import tpu as pltpu
from jax.experimental.pallas import tpu_sc as plsc

B, C, T, D = 16, 16, 200, 768
NC, NS = 2, 16
NW = NC * NS
L = 16
PAIRS_W = B * C // NW
ROWS_W = PAIRS_W * T
R = 40
NCHUNKS = ROWS_W // R
CPP = T // R
NBUF = 4


def _sc_body(x_hbm, ids_hbm, tok_hbm, cht_hbm, out_hbm,
             idx_all, ids_v, ch8,
             b0, b1, b2, b3, g0, g1, g2, g3, o0, o1, o2, o3, psem, csem):
    bufs = (b0, b1, b2, b3)
    gsems = (g0, g1, g2, g3)
    osems = (o0, o1, o2, o3)
    wid = lax.axis_index("s") * NC + lax.axis_index("c")
    bi = wid // 2
    c0 = lax.rem(wid, 2) * PAIRS_W
    idx_cp = pltpu.async_copy(x_hbm.at[pl.ds(wid * ROWS_W, ROWS_W)],
                              idx_all, psem)
    pltpu.sync_copy(ids_hbm, ids_v)
    ch_cp = pltpu.async_copy(cht_hbm.at[ids_v.at[pl.ds(c0, PAIRS_W)]], ch8,
                             csem)
    idx_cp.wait()

    def issue_gather(j, b):
        pltpu.async_copy(tok_hbm.at[idx_all.at[pl.ds(j * R, R)]],
                         bufs[b], gsems[b])

    def wait_gather(b):
        pltpu.make_async_copy(tok_hbm.at[idx_all.at[pl.ds(0, R)]],
                              bufs[b], gsems[b]).wait()

    def issue_scatter(j, b):
        pltpu.async_copy(bufs[b],
                         out_hbm.at[pl.ds(wid * ROWS_W + j * R, R)],
                         osems[b])

    def wait_scatter(b):
        pltpu.make_async_copy(bufs[b], out_hbm.at[pl.ds(0, R)],
                              osems[b]).wait()

    def add_channel(j, b):
        cl = j // CPP
        chvs = [ch8[cl, pl.ds(d * L, L)] for d in range(D // L)]
        buf = bufs[b]

        def row_body(r, rc):
            for d in range(D // L):
                buf[r, pl.ds(d * L, L)] += chvs[d]
            return rc

        lax.fori_loop(0, R, row_body, 0, unroll=False)

    def slot(j, b):
        nb = (b + 2) % NBUF

        @pl.when(j >= 2)
        def _():
            wait_scatter(nb)

        @pl.when(j + 2 < NCHUNKS)
        def _():
            issue_gather(j + 2, nb)

        wait_gather(b)
        add_channel(j, b)
        issue_scatter(j, b)

    issue_gather(0, 0)
    issue_gather(1, 1)
    ch_cp.wait()

    def outer(k, carry):
        j0 = 4 * k
        for s in range(4):
            slot(j0 + s, s)
        return carry

    lax.fori_loop(0, NCHUNKS // 4, outer, 0, unroll=False)

    wait_scatter((NCHUNKS - 2) % NBUF)
    wait_scatter((NCHUNKS - 1) % NBUF)


@jax.jit
def _sc_call(x32, ids32, token_table, channel_table):
    mesh = plsc.VectorSubcoreMesh(core_axis_name="c", subcore_axis_name="s")
    f = pl.kernel(
        _sc_body,
        out_type=jax.ShapeDtypeStruct((B * C * T, D), jnp.float32),
        mesh=mesh,
        scratch_types=[
            pltpu.VMEM((ROWS_W,), jnp.int32),
            pltpu.VMEM((C,), jnp.int32),
            pltpu.VMEM((PAIRS_W, D), jnp.float32),
            pltpu.VMEM((R, D), jnp.float32),
            pltpu.VMEM((R, D), jnp.float32),
            pltpu.VMEM((R, D), jnp.float32),
            pltpu.VMEM((R, D), jnp.float32),
            pltpu.SemaphoreType.DMA,
            pltpu.SemaphoreType.DMA,
            pltpu.SemaphoreType.DMA,
            pltpu.SemaphoreType.DMA,
            pltpu.SemaphoreType.DMA,
            pltpu.SemaphoreType.DMA,
            pltpu.SemaphoreType.DMA,
            pltpu.SemaphoreType.DMA,
            pltpu.SemaphoreType.DMA,
            pltpu.SemaphoreType.DMA,
        ],
    )
    return f(x32, ids32, token_table, channel_table)


def kernel(x, ids, token_table, channel_table):
    out = _sc_call(x.reshape(-1).astype(jnp.int32), ids.astype(jnp.int32),
                   token_table, channel_table)
    return out.reshape(B, C, T, D)


# speedup vs baseline: 1.0360x; 1.0009x over previous
import jax
import jax.numpy as jnp
from jax import lax
from jax.experimental import pallas as pl
from jax.experimental.pallas import tpu as pltpu
from jax.experimental.pallas import tpu_sc as plsc

B, C, T, D = 16, 16, 200, 768
NC, NS = 2, 16
NW = NC * NS
L = 16
PAIRS_W = B * C // NW
ROWS_W = PAIRS_W * T
R = 40
NCHUNKS = ROWS_W // R
CPP = T // R
NBUF = 4


def _sc_body(x_hbm, ids_hbm, tok_hbm, cht_hbm, out_hbm,
             idx_all, ids_v, ch8,
             b0, b1, b2, b3, g0, g1, g2, g3, o0, o1, o2, o3, psem, csem):
    bufs = (b0, b1, b2, b3)
    gsems = (g0, g1, g2, g3)
    osems = (o0, o1, o2, o3)
    wid = lax.axis_index("s") * NC + lax.axis_index("c")
    c0 = lax.rem(wid, 2) * PAIRS_W
    idx_cp = pltpu.async_copy(x_hbm.at[pl.ds(wid * ROWS_W, ROWS_W)],
                              idx_all, psem)
    pltpu.sync_copy(ids_hbm, ids_v)
    ch_cp = pltpu.async_copy(cht_hbm.at[ids_v.at[pl.ds(c0, PAIRS_W)]], ch8,
                             csem)
    idx_cp.wait()

    def issue_gather(j, b):
        pltpu.async_copy(tok_hbm.at[idx_all.at[pl.ds(j * R, R)]],
                         bufs[b], gsems[b])

    def wait_gather(b):
        pltpu.make_async_copy(tok_hbm.at[idx_all.at[pl.ds(0, R)]],
                              bufs[b], gsems[b]).wait()

    def issue_scatter(j, b):
        pltpu.async_copy(bufs[b],
                         out_hbm.at[pl.ds(wid * ROWS_W + j * R, R)],
                         osems[b])

    def wait_scatter(b):
        pltpu.make_async_copy(bufs[b], out_hbm.at[pl.ds(0, R)],
                              osems[b]).wait()

    def add_channel(j, b):
        cl = j // CPP
        chvs = [ch8[cl, pl.ds(d * L, L)] for d in range(D // L)]
        buf = bufs[b]

        def row_body(r, rc):
            for d in range(D // L):
                buf[r, pl.ds(d * L, L)] += chvs[d]
            return rc

        lax.fori_loop(0, R, row_body, 0, unroll=False)

    def slot(j, b):
        nb = (b + 2) % NBUF

        @pl.when(j >= 2)
        def _():
            wait_scatter(nb)

        @pl.when(j + 2 < NCHUNKS)
        def _():
            issue_gather(j + 2, nb)

        wait_gather(b)
        add_channel(j, b)
        issue_scatter(j, b)

    issue_gather(0, 0)
    issue_gather(1, 1)
    ch_cp.wait()

    def outer(k, carry):
        j0 = 4 * k
        for s in range(4):
            slot(j0 + s, s)
        return carry

    lax.fori_loop(0, NCHUNKS // 4, outer, 0, unroll=False)

    wait_scatter((NCHUNKS - 2) % NBUF)
    wait_scatter((NCHUNKS - 1) % NBUF)


@jax.jit
def _sc_call(x32, ids32, token_table, channel_table):
    mesh = plsc.VectorSubcoreMesh(core_axis_name="c", subcore_axis_name="s")
    f = pl.kernel(
        _sc_body,
        out_type=jax.ShapeDtypeStruct((B * C * T, D), jnp.float32),
        mesh=mesh,
        scratch_types=[
            pltpu.VMEM((ROWS_W,), jnp.int32),
            pltpu.VMEM((C,), jnp.int32),
            pltpu.VMEM((PAIRS_W, D), jnp.float32),
            pltpu.VMEM((R, D), jnp.float32),
            pltpu.VMEM((R, D), jnp.float32),
            pltpu.VMEM((R, D), jnp.float32),
            pltpu.VMEM((R, D), jnp.float32),
            pltpu.SemaphoreType.DMA,
            pltpu.SemaphoreType.DMA,
            pltpu.SemaphoreType.DMA,
            pltpu.SemaphoreType.DMA,
            pltpu.SemaphoreType.DMA,
            pltpu.SemaphoreType.DMA,
            pltpu.SemaphoreType.DMA,
            pltpu.SemaphoreType.DMA,
            pltpu.SemaphoreType.DMA,
            pltpu.SemaphoreType.DMA,
        ],
    )
    return f(x32, ids32, token_table, channel_table)


def kernel(x, ids, token_table, channel_table):
    out = _sc_call(x.reshape(-1).astype(jnp.int32), ids.astype(jnp.int32),
                   token_table, channel_table)
    return out.reshape(B, C, T, D)
